# baseline (speedup 1.0000x reference)
import jax
import jax.numpy as jnp
from jax import lax
from jax.experimental import pallas as pl
from jax.experimental.pallas import tpu as pltpu
from jax.experimental.pallas import tpu_sc as plsc

BATCH = 4
SEQ = 8192
D = 1024
SC_ROWS = SEQ // 4
TC_A_ROWS = SEQ - SC_ROWS

NW = 32
CHUNK = 32
ROWS_PER_W = SC_ROWS // NW
NCHUNK = ROWS_PER_W // CHUNK

SEQ_BLOCK = 2048
NS_A = TC_A_ROWS // SEQ_BLOCK
NS_B = SC_ROWS // SEQ_BLOCK


def _sc_gather_body(tab_hbm, pos_hbm, out_hbm, pos_v, buf_a, buf_b, buf_c,
                    gsem_a, gsem_b, gsem_c, ssem_a, ssem_b, ssem_c):
    info = plsc.get_sparse_core_info()
    wid = lax.axis_index("s") * info.num_cores + lax.axis_index("c")
    off = TC_A_ROWS + wid * ROWS_PER_W
    out0 = wid * ROWS_PER_W

    bufs = (buf_a, buf_b, buf_c)
    gsems = (gsem_a, gsem_b, gsem_c)
    ssems = (ssem_a, ssem_b, ssem_c)

    pltpu.sync_copy(pos_hbm.at[pl.ds(off, ROWS_PER_W)], pos_v)

    gathers = [None] * NCHUNK
    stores = [None] * NCHUNK

    def start_gather(ci):
        p = ci % 3
        gathers[ci] = pltpu.async_copy(
            tab_hbm.at[pos_v.at[pl.ds(ci * CHUNK, CHUNK)]], bufs[p], gsems[p])

    start_gather(0)
    start_gather(1)
    for ci in range(NCHUNK):
        p = ci % 3
        if ci + 2 < NCHUNK:
            if ci - 1 >= 0:
                stores[ci - 1].wait()
            start_gather(ci + 2)
        gathers[ci].wait()
        stores[ci] = pltpu.async_copy(
            bufs[p], out_hbm.at[pl.ds(out0 + ci * CHUNK, CHUNK)], ssems[p])
    for st in stores[max(0, NCHUNK - 3):]:
        st.wait()


def _sc_gather_half(pos_table, pos32):
    mesh = plsc.VectorSubcoreMesh(core_axis_name="c", subcore_axis_name="s")
    return pl.kernel(
        _sc_gather_body,
        out_type=jax.ShapeDtypeStruct((SC_ROWS, D), jnp.float32),
        mesh=mesh,
        scratch_types=[
            pltpu.VMEM((ROWS_PER_W,), jnp.int32),
            pltpu.VMEM((CHUNK, D), jnp.float32),
            pltpu.VMEM((CHUNK, D), jnp.float32),
            pltpu.VMEM((CHUNK, D), jnp.float32),
            pltpu.SemaphoreType.DMA,
            pltpu.SemaphoreType.DMA,
            pltpu.SemaphoreType.DMA,
            pltpu.SemaphoreType.DMA,
            pltpu.SemaphoreType.DMA,
            pltpu.SemaphoreType.DMA,
        ],
    )(pos_table, pos32)


def _tc_kernel(pos_ref, x_ref, tab_ref, emb_ref, out_ref):
    s = pl.program_id(0)

    @pl.when(s < NS_A)
    def _():
        out_ref[...] = x_ref[...] + tab_ref[...]

    @pl.when(s >= NS_A)
    def _():
        out_ref[...] = x_ref[...] + emb_ref[...]


def _tc_add(pos32, x, pos_table, emb_b):
    ns = SEQ // SEQ_BLOCK
    grid_spec = pltpu.PrefetchScalarGridSpec(
        num_scalar_prefetch=1,
        grid=(ns, BATCH),
        in_specs=[
            pl.BlockSpec((1, SEQ_BLOCK, D), lambda s, b, pos: (b, s, 0)),
            pl.BlockSpec(
                (SEQ_BLOCK, D),
                lambda s, b, pos: (pos[jnp.minimum(s, NS_A - 1) * SEQ_BLOCK]
                                   // SEQ_BLOCK, 0),
            ),
            pl.BlockSpec((SEQ_BLOCK, D), lambda s, b, pos: (0, 0)),
        ],
        out_specs=pl.BlockSpec((1, SEQ_BLOCK, D), lambda s, b, pos: (b, s, 0)),
    )
    return pl.pallas_call(
        _tc_kernel,
        grid_spec=grid_spec,
        out_shape=jax.ShapeDtypeStruct((BATCH, SEQ, D), jnp.float32),
        compiler_params=pltpu.CompilerParams(
            dimension_semantics=("arbitrary", "arbitrary"),
        ),
    )(pos32, x, pos_table, emb_b)


def kernel(x, pos_table, positions):
    pos32 = positions.astype(jnp.int32)
    emb_b = _sc_gather_half(pos_table, pos32)
    return _tc_add(pos32, x, pos_table, emb_b)


# speedup vs baseline: 1.0365x; 1.0365x over previous
import jax
import jax.numpy as jnp
from jax import lax
from jax.experimental import pallas as pl
from jax.experimental.pallas import tpu as pltpu
from jax.experimental.pallas import tpu_sc as plsc

BATCH = 4
SEQ = 8192
D = 1024
SC_ROWS = SEQ // 4
TC_A_ROWS = SEQ - SC_ROWS

NW = 32
CHUNK = 32
ROWS_PER_W = SC_ROWS // NW
NCHUNK = ROWS_PER_W // CHUNK

SEQ_BLOCK = 2048
NS_A = TC_A_ROWS // SEQ_BLOCK
NS_B = SC_ROWS // SEQ_BLOCK


def _sc_gather_body(tab_hbm, pos_hbm, out_hbm, pos_v, buf_a, buf_b, buf_c,
                    gsem_a, gsem_b, gsem_c, ssem_a, ssem_b, ssem_c):
    info = plsc.get_sparse_core_info()
    wid = lax.axis_index("s") * info.num_cores + lax.axis_index("c")
    off = TC_A_ROWS + wid * ROWS_PER_W
    out0 = wid * ROWS_PER_W

    bufs = (buf_a, buf_b, buf_c)
    gsems = (gsem_a, gsem_b, gsem_c)
    ssems = (ssem_a, ssem_b, ssem_c)

    pltpu.sync_copy(pos_hbm.at[pl.ds(off, ROWS_PER_W)], pos_v)

    gathers = [None] * NCHUNK
    stores = [None] * NCHUNK

    def start_gather(ci):
        p = ci % 3
        gathers[ci] = pltpu.async_copy(
            tab_hbm.at[pos_v.at[pl.ds(ci * CHUNK, CHUNK)]], bufs[p], gsems[p])

    start_gather(0)
    start_gather(1)
    for ci in range(NCHUNK):
        p = ci % 3
        if ci + 2 < NCHUNK:
            if ci - 1 >= 0:
                stores[ci - 1].wait()
            start_gather(ci + 2)
        gathers[ci].wait()
        stores[ci] = pltpu.async_copy(
            bufs[p], out_hbm.at[pl.ds(out0 + ci * CHUNK, CHUNK)], ssems[p])
    for st in stores[max(0, NCHUNK - 3):]:
        st.wait()


def _sc_gather_half(pos_table, pos32):
    mesh = plsc.VectorSubcoreMesh(core_axis_name="c", subcore_axis_name="s")
    return pl.kernel(
        _sc_gather_body,
        out_type=jax.ShapeDtypeStruct((SC_ROWS, D), jnp.float32),
        mesh=mesh,
        scratch_types=[
            pltpu.VMEM((ROWS_PER_W,), jnp.int32),
            pltpu.VMEM((CHUNK, D), jnp.float32),
            pltpu.VMEM((CHUNK, D), jnp.float32),
            pltpu.VMEM((CHUNK, D), jnp.float32),
            pltpu.SemaphoreType.DMA,
            pltpu.SemaphoreType.DMA,
            pltpu.SemaphoreType.DMA,
            pltpu.SemaphoreType.DMA,
            pltpu.SemaphoreType.DMA,
            pltpu.SemaphoreType.DMA,
        ],
    )(pos_table, pos32)


def _tc_a_kernel(pos_ref, x_ref, tab_ref, out_ref):
    out_ref[...] = x_ref[...] + tab_ref[...]


def _tc_stage_a(pos32, x, pos_table):
    grid_spec = pltpu.PrefetchScalarGridSpec(
        num_scalar_prefetch=1,
        grid=(NS_A, BATCH),
        in_specs=[
            pl.BlockSpec((1, SEQ_BLOCK, D), lambda s, b, pos: (b, s, 0)),
            pl.BlockSpec(
                (SEQ_BLOCK, D),
                lambda s, b, pos: (pos[s * SEQ_BLOCK] // SEQ_BLOCK, 0),
            ),
        ],
        out_specs=pl.BlockSpec((1, SEQ_BLOCK, D), lambda s, b, pos: (b, s, 0)),
    )
    return pl.pallas_call(
        _tc_a_kernel,
        grid_spec=grid_spec,
        out_shape=jax.ShapeDtypeStruct((BATCH, SEQ, D), jnp.float32),
        compiler_params=pltpu.CompilerParams(
            dimension_semantics=("arbitrary", "arbitrary"),
        ),
    )(pos32, x, pos_table)


def _tc_b_kernel(x_ref, emb_ref, prev_ref, out_ref):
    out_ref[...] = x_ref[...] + emb_ref[...]


def _tc_stage_b(x, emb_b, prev):
    return pl.pallas_call(
        _tc_b_kernel,
        grid=(NS_B, BATCH),
        in_specs=[
            pl.BlockSpec((1, SEQ_BLOCK, D), lambda s, b: (b, s + NS_A, 0)),
            pl.BlockSpec((SEQ_BLOCK, D), lambda s, b: (s, 0)),
            pl.BlockSpec(memory_space=pltpu.MemorySpace.HBM),
        ],
        out_specs=pl.BlockSpec((1, SEQ_BLOCK, D), lambda s, b: (b, s + NS_A, 0)),
        out_shape=jax.ShapeDtypeStruct((BATCH, SEQ, D), jnp.float32),
        input_output_aliases={2: 0},
        compiler_params=pltpu.CompilerParams(
            dimension_semantics=("arbitrary", "arbitrary"),
        ),
    )(x, emb_b, prev)


def kernel(x, pos_table, positions):
    pos32 = positions.astype(jnp.int32)
    emb_b = _sc_gather_half(pos_table, pos32)
    out_a = _tc_stage_a(pos32, x, pos_table)
    return _tc_stage_b(x, emb_b, out_a)
